# baseline (speedup 1.0000x reference)
import functools

import jax
import jax.numpy as jnp
from jax import lax
from jax.experimental import pallas as pl
from jax.experimental.pallas import tpu as pltpu
from jax.experimental.pallas import tpu_sc as plsc

_BATCH = 4
_SEQ = 4096
_DIM = 1024
_NC = 2
_NS = 16
_NW = _NC * _NS
_ROWS_PER_W = _SEQ // _NW
_CHUNK = 16
_NCHUNK = _ROWS_PER_W // _CHUNK
_NBUF = 6


@functools.partial(
    pl.kernel,
    mesh=plsc.VectorSubcoreMesh(core_axis_name="c", subcore_axis_name="s"),
    out_type=jax.ShapeDtypeStruct((_BATCH, _SEQ, _DIM), jnp.float32),
    scratch_types=[
        pltpu.VMEM((_NBUF, _CHUNK, _DIM), jnp.float32),
        pltpu.SemaphoreType.DMA,
        pltpu.SemaphoreType.DMA,
        pltpu.SemaphoreType.DMA,
        pltpu.SemaphoreType.DMA,
        pltpu.SemaphoreType.DMA,
        pltpu.SemaphoreType.DMA,
        pltpu.SemaphoreType.DMA,
    ],
)
def _broadcast_rows(w_hbm, out_hbm, ring, rsem, wsem_0, wsem_1, wsem_2, wsem_3, wsem_4, wsem_5):
    wid = lax.axis_index("s") * _NC + lax.axis_index("c")
    base = wid * _ROWS_PER_W
    wsems = (wsem_0, wsem_1, wsem_2, wsem_3, wsem_4, wsem_5)

    def row_slice(i):
        return pl.ds(base + i * _CHUNK, _CHUNK)

    reads = []
    cp = pltpu.make_async_copy(w_hbm.at[row_slice(0)], ring.at[0], rsem)
    cp.start()
    reads.append(cp)

    writes = []
    for i in range(_NCHUNK):
        slot = i % _NBUF
        nxt = i + 1
        if nxt < _NCHUNK:
            if nxt >= _NBUF:
                for cp in writes[nxt - _NBUF]:
                    cp.wait()
            cp = pltpu.make_async_copy(
                w_hbm.at[row_slice(nxt)], ring.at[nxt % _NBUF], rsem
            )
            cp.start()
            reads.append(cp)
        reads[i].wait()
        cps = [
            pltpu.make_async_copy(ring.at[slot], out_hbm.at[b].at[row_slice(i)], wsems[slot])
            for b in range(_BATCH)
        ]
        for cp in cps:
            cp.start()
        writes.append(cps)
    for i in range(max(0, _NCHUNK - _NBUF), _NCHUNK):
        for cp in writes[i]:
            cp.wait()


def kernel(input, weights):
    del input
    return _broadcast_rows(weights)


# speedup vs baseline: 1.0460x; 1.0460x over previous
import functools

import jax
import jax.numpy as jnp
from jax import lax
from jax.experimental import pallas as pl
from jax.experimental.pallas import tpu as pltpu
from jax.experimental.pallas import tpu_sc as plsc

_BATCH = 4
_SEQ = 4096
_DIM = 1024
_NC = 2
_NS = 16
_NW = _NC * _NS
_ROWS_PER_W = _SEQ // _NW
_CHUNK = 32
_NCHUNK = _ROWS_PER_W // _CHUNK
_NBUF = 3


@functools.partial(
    pl.kernel,
    mesh=plsc.VectorSubcoreMesh(core_axis_name="c", subcore_axis_name="s"),
    out_type=jax.ShapeDtypeStruct((_BATCH, _SEQ, _DIM), jnp.float32),
    scratch_types=[
        pltpu.VMEM((_NBUF, _CHUNK, _DIM), jnp.float32),
        pltpu.SemaphoreType.DMA,
        pltpu.SemaphoreType.DMA,
        pltpu.SemaphoreType.DMA,
        pltpu.SemaphoreType.DMA,
    ],
)
def _broadcast_rows(w_hbm, out_hbm, ring, rsem, wsem_0, wsem_1, wsem_2):
    wid = lax.axis_index("s") * _NC + lax.axis_index("c")
    base = wid * _ROWS_PER_W
    wsems = (wsem_0, wsem_1, wsem_2)

    def row_slice(i):
        return pl.ds(base + i * _CHUNK, _CHUNK)

    reads = []
    cp = pltpu.make_async_copy(w_hbm.at[row_slice(0)], ring.at[0], rsem)
    cp.start()
    reads.append(cp)

    writes = []
    for i in range(_NCHUNK):
        slot = i % _NBUF
        nxt = i + 1
        if nxt < _NCHUNK:
            if nxt >= _NBUF:
                for cp in writes[nxt - _NBUF]:
                    cp.wait()
            cp = pltpu.make_async_copy(
                w_hbm.at[row_slice(nxt)], ring.at[nxt % _NBUF], rsem
            )
            cp.start()
            reads.append(cp)
        reads[i].wait()
        cps = [
            pltpu.make_async_copy(ring.at[slot], out_hbm.at[b].at[row_slice(i)], wsems[slot])
            for b in range(_BATCH)
        ]
        for cp in cps:
            cp.start()
        writes.append(cps)
    for i in range(max(0, _NCHUNK - _NBUF), _NCHUNK):
        for cp in writes[i]:
            cp.wait()


def kernel(input, weights):
    del input
    return _broadcast_rows(weights)
